# Initial kernel scaffold; baseline (speedup 1.0000x reference)
import jax
import jax.numpy as jnp
from jax.experimental import pallas as pl


def kernel(x, embedding, segment_index):
    raise NotImplementedError("write your pallas kernel here")


# speedup vs baseline: 1.0002x; 1.0002x over previous
import jax
import jax.numpy as jnp
from jax.experimental import pallas as pl
from jax.experimental.pallas import tpu as pltpu

_BLOCK_ROWS = 1024


def _body(idx_ref, emb_ref, x_ref, o_ref):
    o_ref[...] = x_ref[...] + emb_ref[0]


def kernel(x, embedding, segment_index):
    B, S, D = x.shape
    rows = B * S
    x2 = x.reshape(rows, D)
    idx = jnp.asarray(segment_index, jnp.int32).reshape(1)

    grid = (rows // _BLOCK_ROWS,)
    out = pl.pallas_call(
        _body,
        grid_spec=pltpu.PrefetchScalarGridSpec(
            num_scalar_prefetch=1,
            grid=grid,
            in_specs=[
                pl.BlockSpec((1, 1, D), lambda i, idx_ref: (idx_ref[0], 0, 0)),
                pl.BlockSpec((_BLOCK_ROWS, D), lambda i, idx_ref: (i, 0)),
            ],
            out_specs=pl.BlockSpec((_BLOCK_ROWS, D), lambda i, idx_ref: (i, 0)),
        ),
        out_shape=jax.ShapeDtypeStruct((rows, D), x.dtype),
    )(idx, embedding, x2)
    return out.reshape(B, S, D)
